# baseline (speedup 1.0000x reference)
import functools

import jax
import jax.numpy as jnp
from jax import lax
from jax.experimental import pallas as pl
from jax.experimental.pallas import tpu as pltpu
from jax.experimental.pallas import tpu_sc as plsc

NUM_GRAPHS = 64
D = 512
N = 100000
NC = 2
NS = 16
NW = NC * NS
L = 16
TILE = 80
NT = N // TILE
DJ = D // L
ACC_ROWS = NUM_GRAPHS * DJ


def _sc_body(x_hbm, b_hbm, watt_hbm, part_hbm, xbuf, bbuf, wbuf, acc):
    wid = lax.axis_index("s") * NC + lax.axis_index("c")
    base = NT // NW
    extra = NT % NW
    start = wid * base + jnp.minimum(wid, extra)
    count = base + (wid < extra).astype(jnp.int32)

    pltpu.sync_copy(watt_hbm.at[0], wbuf)

    def init_body(k, c):
        acc[k] = jnp.full((L,), -jnp.inf, jnp.float32)
        return c

    lax.fori_loop(0, ACC_ROWS, init_body, 0)

    def tile_body(t, c):
        row0 = t * TILE
        pltpu.sync_copy(x_hbm.at[pl.ds(row0, TILE)], xbuf)
        pltpu.sync_copy(b_hbm.at[pl.ds(row0, TILE)], bbuf.at[pl.ds(0, TILE)])

        def row_body(r, c2):
            att = jnp.zeros((L,), jnp.float32)
            for j in range(DJ):
                att = att + xbuf[r, pl.ds(j * L, L)] * wbuf[pl.ds(j * L, L)]
            a = plsc.cumsum(att)[L - 1]
            av = jnp.full((L,), a, jnp.float32)
            scale = (1.0 / (1.0 + jnp.exp(-av)) + 1.0) * 0.5
            seg = bbuf[pl.ds(r, L)][0]
            k0 = seg * DJ
            for j in range(DJ):
                yv = xbuf[r, pl.ds(j * L, L)] * scale
                acc[k0 + j] = jnp.maximum(acc[k0 + j], yv)
            return c2

        lax.fori_loop(0, TILE, row_body, 0)
        return c

    lax.fori_loop(start, start + count, tile_body, 0)
    pltpu.sync_copy(acc, part_hbm.at[wid])


def _sc_partials(x, batch, W_att):
    mesh = plsc.VectorSubcoreMesh(
        core_axis_name="c", subcore_axis_name="s",
        num_cores=NC, num_subcores=NS)
    f = pl.kernel(
        _sc_body,
        out_type=jax.ShapeDtypeStruct((NW, ACC_ROWS, L), jnp.float32),
        mesh=mesh,
        compiler_params=pltpu.CompilerParams(needs_layout_passes=False, use_tc_tiling_on_sc=False),
        scratch_types=[
            pltpu.VMEM((TILE, D), jnp.float32),
            pltpu.VMEM((TILE + L,), jnp.int32),
            pltpu.VMEM((D,), jnp.float32),
            pltpu.VMEM((ACC_ROWS, L), jnp.float32),
        ],
    )
    return f(x, batch, W_att)


def _merge_body(part_ref, wout_ref, out_ref):
    def body(w, m):
        return jnp.maximum(m, part_ref[w])

    hg = lax.fori_loop(1, NW, body, part_ref[0])
    out_ref[...] = jax.lax.dot_general(
        hg, wout_ref[...], (((1,), (1,)), ((), ())),
        preferred_element_type=jnp.float32)


@jax.jit
def kernel(x, batch, W_att, W_out):
    n_classes = W_out.shape[0]
    part = _sc_partials(x, batch.astype(jnp.int32), W_att)
    part = part.reshape(NW, NUM_GRAPHS, D)
    return pl.pallas_call(
        _merge_body,
        in_specs=[
            pl.BlockSpec((NW, NUM_GRAPHS, D), lambda: (0, 0, 0)),
            pl.BlockSpec((n_classes, D), lambda: (0, 0)),
        ],
        out_specs=pl.BlockSpec((NUM_GRAPHS, n_classes), lambda: (0, 0)),
        out_shape=jax.ShapeDtypeStruct((NUM_GRAPHS, n_classes), jnp.float32),
    )(part, W_out)


# speedup vs baseline: 6.4572x; 6.4572x over previous
import functools

import jax
import jax.numpy as jnp
from jax.experimental import pallas as pl
from jax.experimental.pallas import tpu as pltpu

NUM_GRAPHS = 64
BLOCK_ROWS = 4000


def _body(lo_ref, hi_ref, x_ref, b_ref, watt_ref, wout_ref, out_ref, hg_ref):
    i = pl.program_id(0)
    nb = pl.num_programs(0)

    @pl.when(i == 0)
    def _init():
        hg_ref[...] = jnp.full_like(hg_ref, -jnp.inf)

    xb = x_ref[...]
    att = jax.lax.dot_general(
        xb, watt_ref[...], (((1,), (1,)), ((), ())),
        preferred_element_type=jnp.float32)
    scale = (jax.nn.sigmoid(att) + 1.0) * 0.5
    y = xb * scale

    bcol = b_ref[0]
    s_lo = lo_ref[i]
    s_hi = hi_ref[i]

    def seg_body(s, carry):
        m = bcol == s
        col = jnp.max(jnp.where(m, y, -jnp.inf), axis=0,
                      keepdims=True)
        hg_ref[pl.ds(s, 1), :] = jnp.maximum(hg_ref[pl.ds(s, 1), :], col)
        return carry

    jax.lax.fori_loop(s_lo, s_hi + 1, seg_body, 0)

    @pl.when(i == nb - 1)
    def _readout():
        out_ref[...] = jax.lax.dot_general(
            hg_ref[...], wout_ref[...], (((1,), (1,)), ((), ())),
            preferred_element_type=jnp.float32)


@jax.jit
def kernel(x, batch, W_att, W_out):
    n, d = x.shape
    n_classes = W_out.shape[0]
    b = BLOCK_ROWS
    nb = n // b
    batch = batch.astype(jnp.int32)
    batch_r = batch.reshape(nb, b, 1)
    blk_lo = batch[::b]
    blk_hi = batch[b - 1::b]

    grid_spec = pltpu.PrefetchScalarGridSpec(
        num_scalar_prefetch=2,
        grid=(nb,),
        in_specs=[
            pl.BlockSpec((b, d), lambda i, lo, hi: (i, 0)),
            pl.BlockSpec((1, b, 1), lambda i, lo, hi: (i, 0, 0)),
            pl.BlockSpec((1, d), lambda i, lo, hi: (0, 0)),
            pl.BlockSpec((n_classes, d), lambda i, lo, hi: (0, 0)),
        ],
        out_specs=pl.BlockSpec((NUM_GRAPHS, n_classes),
                               lambda i, lo, hi: (0, 0)),
        scratch_shapes=[pltpu.VMEM((NUM_GRAPHS, d), jnp.float32)],
    )

    return pl.pallas_call(
        _body,
        grid_spec=grid_spec,
        out_shape=jax.ShapeDtypeStruct((NUM_GRAPHS, n_classes), jnp.float32),
    )(blk_lo, blk_hi, x, batch_r, W_att, W_out)
